# baseline (speedup 1.0000x reference)
import functools
import jax
import jax.numpy as jnp
from jax import lax
from jax.experimental import pallas as pl
from jax.experimental.pallas import tpu as pltpu
from jax.experimental.pallas import tpu_sc as plsc

_NUM_MODALITIES = 3
_HIDDEN = 1024
_N = 4 * 8192
_NC, _NS = 2, 16
_NW = _NC * _NS
_RPW = _N // _NW
_K = 64
_NG = _RPW // _K

_mesh = plsc.VectorSubcoreMesh(core_axis_name="c", subcore_axis_name="s")


@functools.partial(
    pl.kernel,
    out_type=jax.ShapeDtypeStruct((_N, _HIDDEN), jnp.float32),
    mesh=_mesh,
    scratch_types=[
        pltpu.VMEM((_RPW,), jnp.int32),
        pltpu.VMEM((_NUM_MODALITIES, _HIDDEN), jnp.float32),
        pltpu.SemaphoreType.DMA,
    ],
)
def _emb_lookup(ids_hbm, table_hbm, out_hbm, idx_v, table_v, dsem):
    wid = lax.axis_index("s") * _NC + lax.axis_index("c")
    base = wid * _RPW
    pltpu.sync_copy(table_hbm, table_v)
    pltpu.sync_copy(ids_hbm.at[pl.ds(base, _RPW)], idx_v)

    @pl.loop(0, _NG)
    def _grp(g):
        r0 = g * _K
        for v0 in range(0, _K, 16):
            ids_vec = idx_v[pl.ds(r0 + v0, 16)]
            for i in range(16):
                m = ids_vec[i]
                pltpu.async_copy(
                    table_v.at[m], out_hbm.at[base + r0 + v0 + i], dsem
                )

        @pl.when(g > 0)
        def _():
            for _ in range(_K):
                pltpu.make_async_copy(table_v.at[0], out_hbm.at[base], dsem).wait()

    for _ in range(_K):
        pltpu.make_async_copy(table_v.at[0], out_hbm.at[base], dsem).wait()


def kernel(modality_ids, emb_weight):
    ids_flat = modality_ids.reshape(-1).astype(jnp.int32)
    out = _emb_lookup(ids_flat, emb_weight)
    return out.reshape(modality_ids.shape + (_HIDDEN,))


# speedup vs baseline: 1.0441x; 1.0441x over previous
import functools
import jax
import jax.numpy as jnp
from jax import lax
from jax.experimental import pallas as pl
from jax.experimental.pallas import tpu as pltpu
from jax.experimental.pallas import tpu_sc as plsc

_NUM_MODALITIES = 3
_HIDDEN = 1024
_N = 4 * 8192
_NC, _NS = 2, 16
_NW = _NC * _NS
_RPW = _N // _NW
_K = 16
_NG = _RPW // _K

_mesh = plsc.VectorSubcoreMesh(core_axis_name="c", subcore_axis_name="s")


@functools.partial(
    pl.kernel,
    out_type=jax.ShapeDtypeStruct((_N, _HIDDEN), jnp.float32),
    mesh=_mesh,
    scratch_types=[
        pltpu.VMEM((_RPW,), jnp.int32),
        pltpu.VMEM((_NUM_MODALITIES, _HIDDEN), jnp.float32),
        pltpu.SemaphoreType.DMA,
    ],
)
def _emb_lookup(ids_hbm, table_hbm, out_hbm, idx_v, table_v, dsem):
    wid = lax.axis_index("s") * _NC + lax.axis_index("c")
    base = wid * _RPW
    pltpu.sync_copy(table_hbm, table_v)
    pltpu.sync_copy(ids_hbm.at[pl.ds(base, _RPW)], idx_v)

    @pl.loop(0, _NG)
    def _grp(g):
        r0 = g * _K
        for v0 in range(0, _K, 16):
            ids_vec = idx_v[pl.ds(r0 + v0, 16)]
            for i in range(16):
                m = ids_vec[i]
                pltpu.async_copy(
                    table_v.at[m], out_hbm.at[base + r0 + v0 + i], dsem
                )

        @pl.when(g > 0)
        def _():
            for _ in range(_K):
                pltpu.make_async_copy(table_v.at[0], out_hbm.at[base], dsem).wait()

    for _ in range(_K):
        pltpu.make_async_copy(table_v.at[0], out_hbm.at[base], dsem).wait()


def kernel(modality_ids, emb_weight):
    ids_flat = modality_ids.reshape(-1).astype(jnp.int32)
    out = _emb_lookup(ids_flat, emb_weight)
    return out.reshape(modality_ids.shape + (_HIDDEN,))
